# baseline (speedup 1.0000x reference)
import functools

import jax
import jax.numpy as jnp
from jax import lax
from jax.experimental import pallas as pl
from jax.experimental.pallas import tpu as pltpu
from jax.experimental.pallas import tpu_sc as plsc

VOCAB = 100000
EMBED = 128
BATCH = 16384
HIST = 200
HID = 64
LANES = 16

NUM_CORES = 2
NUM_SUBCORES = 16
NW = NUM_CORES * NUM_SUBCORES
ROWS_PER_W = BATCH // NW
IDX_CHUNK_ROWS = 128
N_CHUNKS = ROWS_PER_W // IDX_CHUNK_ROWS
NBUF = 4
GATHER_A = 104
GATHER_B = HIST - GATHER_A


def _proj_body(t_ref, w_ref, o_ref):
    o_ref[...] = lax.dot_general(
        t_ref[...], w_ref[...], (((1,), (1,)), ((), ())),
        precision=lax.Precision.HIGHEST)


def _project_table(table, W1):
    rows_blk = 1000
    return pl.pallas_call(
        _proj_body,
        grid=(VOCAB // rows_blk,),
        in_specs=[
            pl.BlockSpec((rows_blk, EMBED), lambda i: (i, 0)),
            pl.BlockSpec((HID, EMBED), lambda i: (0, 0)),
        ],
        out_specs=pl.BlockSpec((rows_blk, HID), lambda i: (i, 0)),
        out_shape=jax.ShapeDtypeStruct((VOCAB, HID), jnp.float32),
    )(table, W1)


def _sc_body(xf_hbm, t1_hbm, out_hbm, idx_v, rows_v, out_v, sem):
    cid = lax.axis_index("c")
    sid = lax.axis_index("s")
    wid = sid * NUM_CORES + cid
    base = wid * ROWS_PER_W * HIST

    def load_chunk(c):
        pltpu.sync_copy(
            xf_hbm.at[pl.ds(base + c * (IDX_CHUNK_ROWS * HIST),
                            IDX_CHUNK_ROWS * HIST)],
            idx_v)

    def issue(lr, b):
        off = lr * HIST
        pltpu.async_copy(
            t1_hbm.at[idx_v.at[pl.ds(off, GATHER_A)]],
            rows_v.at[b, pl.ds(0, GATHER_A)], sem)
        pltpu.async_copy(
            t1_hbm.at[idx_v.at[pl.ds(off + GATHER_A, GATHER_B)]],
            rows_v.at[b, pl.ds(GATHER_A, GATHER_B)], sem)

    def wait(b):
        pltpu.make_async_copy(
            t1_hbm.at[pl.ds(0, GATHER_A)],
            rows_v.at[b, pl.ds(0, GATHER_A)], sem).wait()
        pltpu.make_async_copy(
            t1_hbm.at[pl.ds(0, GATHER_B)],
            rows_v.at[b, pl.ds(GATHER_A, GATHER_B)], sem).wait()

    def accum_store(c, lr, b):
        def acc_body(i, accs):
            return tuple(a + rows_v[b, i, pl.ds(LANES * k, LANES)]
                         for k, a in enumerate(accs))

        accs = plsc.parallel_loop(
            0, HIST, 1, unroll=8,
            carry=tuple(jnp.zeros((LANES,), jnp.float32)
                        for _ in range(HID // LANES)))(acc_body)
        orow = c * IDX_CHUNK_ROWS + lr
        for k in range(HID // LANES):
            out_v[orow, pl.ds(LANES * k, LANES)] = accs[k]

    def chunk_body(c, carry):
        load_chunk(c)
        for b in range(NBUF - 1):
            issue(b, b)

        def quad_body(q, carry2):
            for b in range(NBUF):
                lr = NBUF * q + b
                wait(b)
                issue(lr + NBUF - 1, (b + NBUF - 1) % NBUF)
                accum_store(c, lr, b)
            return carry2

        lax.fori_loop(0, IDX_CHUNK_ROWS // NBUF - 1, quad_body, 0)
        tail = IDX_CHUNK_ROWS - NBUF
        wait(tail % NBUF)
        issue(IDX_CHUNK_ROWS - 1, (IDX_CHUNK_ROWS - 1) % NBUF)
        accum_store(c, tail, tail % NBUF)
        for j in range(1, NBUF):
            lr = tail + j
            wait(lr % NBUF)
            accum_store(c, lr, lr % NBUF)
        return carry

    lax.fori_loop(0, N_CHUNKS, chunk_body, 0)
    pltpu.sync_copy(out_v, out_hbm.at[pl.ds(wid * ROWS_PER_W, ROWS_PER_W)])


_sc_pool = functools.partial(
    pl.kernel,
    out_type=jax.ShapeDtypeStruct((BATCH, HID), jnp.float32),
    mesh=plsc.VectorSubcoreMesh(
        core_axis_name="c", subcore_axis_name="s",
        num_cores=NUM_CORES, num_subcores=NUM_SUBCORES),
    scratch_types=[
        pltpu.VMEM((IDX_CHUNK_ROWS * HIST,), jnp.int32),
        pltpu.VMEM((NBUF, HIST, HID), jnp.float32),
        pltpu.VMEM((ROWS_PER_W, HID), jnp.float32),
        pltpu.SemaphoreType.DMA,
    ],
    compiler_params=pltpu.CompilerParams(use_tc_tiling_on_sc=False),
)(_sc_body)


def _tail_body(s_ref, b1_ref, w2_ref, b2_ref, o_ref):
    h = jnp.maximum(s_ref[...] * (1.0 / HIST) + b1_ref[...], 0.0)
    o_ref[...] = lax.dot_general(
        h, w2_ref[...], (((1,), (1,)), ((), ())),
        precision=lax.Precision.HIGHEST) + b2_ref[...]


def _mlp_tail(s, b1, W2, b2):
    rows_blk = 2048
    return pl.pallas_call(
        _tail_body,
        grid=(BATCH // rows_blk,),
        in_specs=[
            pl.BlockSpec((rows_blk, HID), lambda i: (i, 0)),
            pl.BlockSpec((1, HID), lambda i: (0, 0)),
            pl.BlockSpec((2, HID), lambda i: (0, 0)),
            pl.BlockSpec((1, 2), lambda i: (0, 0)),
        ],
        out_specs=pl.BlockSpec((rows_blk, 2), lambda i: (i, 0)),
        out_shape=jax.ShapeDtypeStruct((BATCH, 2), jnp.float32),
    )(s, b1, W2, b2)


def kernel(x, table, W1, b1, W2, b2):
    t1 = _project_table(table, W1)
    xf = x.reshape(BATCH * HIST).astype(jnp.int32)
    s = _sc_pool(xf, t1)
    return _mlp_tail(s, b1.reshape(1, HID), W2, b2.reshape(1, 2))


# speedup vs baseline: 1.0011x; 1.0011x over previous
import functools

import jax
import jax.numpy as jnp
from jax import lax
from jax.experimental import pallas as pl
from jax.experimental.pallas import tpu as pltpu
from jax.experimental.pallas import tpu_sc as plsc

VOCAB = 100000
EMBED = 128
BATCH = 16384
HIST = 200
HID = 64
LANES = 16

NUM_CORES = 2
NUM_SUBCORES = 16
NW = NUM_CORES * NUM_SUBCORES
ROWS_PER_W = BATCH // NW
IDX_CHUNK_ROWS = 128
N_CHUNKS = ROWS_PER_W // IDX_CHUNK_ROWS
NBUF = 4
GATHER_A = 104
GATHER_B = HIST - GATHER_A


def _proj_body(t_ref, w_ref, o_ref):
    o_ref[...] = lax.dot_general(
        t_ref[...], w_ref[...], (((1,), (1,)), ((), ())),
        precision=lax.Precision.HIGHEST)


def _project_table(table, W1):
    rows_blk = 1000
    return pl.pallas_call(
        _proj_body,
        grid=(VOCAB // rows_blk,),
        in_specs=[
            pl.BlockSpec((rows_blk, EMBED), lambda i: (i, 0)),
            pl.BlockSpec((HID, EMBED), lambda i: (0, 0)),
        ],
        out_specs=pl.BlockSpec((rows_blk, HID), lambda i: (i, 0)),
        out_shape=jax.ShapeDtypeStruct((VOCAB, HID), jnp.float32),
    )(table, W1)


def _sc_body(xf_hbm, t1_hbm, out_hbm, idx_v, rows_v, out_v, sem):
    cid = lax.axis_index("c")
    sid = lax.axis_index("s")
    wid = sid * NUM_CORES + cid
    base = wid * ROWS_PER_W * HIST

    def load_chunk(c):
        pltpu.sync_copy(
            xf_hbm.at[pl.ds(base + c * (IDX_CHUNK_ROWS * HIST),
                            IDX_CHUNK_ROWS * HIST)],
            idx_v)

    def issue(lr, b):
        pltpu.async_copy(
            t1_hbm.at[idx_v.at[pl.ds(lr * HIST, HIST)]],
            rows_v.at[b], sem)

    def wait(b):
        pltpu.make_async_copy(
            t1_hbm.at[pl.ds(0, HIST)],
            rows_v.at[b], sem).wait()

    def accum_store(c, lr, b):
        def acc_body(i, accs):
            return tuple(a + rows_v[b, i, pl.ds(LANES * k, LANES)]
                         for k, a in enumerate(accs))

        accs = plsc.parallel_loop(
            0, HIST, 1, unroll=8,
            carry=tuple(jnp.zeros((LANES,), jnp.float32)
                        for _ in range(HID // LANES)))(acc_body)
        orow = c * IDX_CHUNK_ROWS + lr
        for k in range(HID // LANES):
            out_v[orow, pl.ds(LANES * k, LANES)] = accs[k]

    def chunk_body(c, carry):
        load_chunk(c)
        for b in range(NBUF - 1):
            issue(b, b)

        def quad_body(q, carry2):
            for b in range(NBUF):
                lr = NBUF * q + b
                wait(b)
                issue(lr + NBUF - 1, (b + NBUF - 1) % NBUF)
                accum_store(c, lr, b)
            return carry2

        lax.fori_loop(0, IDX_CHUNK_ROWS // NBUF - 1, quad_body, 0)
        tail = IDX_CHUNK_ROWS - NBUF
        wait(tail % NBUF)
        issue(IDX_CHUNK_ROWS - 1, (IDX_CHUNK_ROWS - 1) % NBUF)
        accum_store(c, tail, tail % NBUF)
        for j in range(1, NBUF):
            lr = tail + j
            wait(lr % NBUF)
            accum_store(c, lr, lr % NBUF)
        return carry

    lax.fori_loop(0, N_CHUNKS, chunk_body, 0)
    pltpu.sync_copy(out_v, out_hbm.at[pl.ds(wid * ROWS_PER_W, ROWS_PER_W)])


_sc_pool = functools.partial(
    pl.kernel,
    out_type=jax.ShapeDtypeStruct((BATCH, HID), jnp.float32),
    mesh=plsc.VectorSubcoreMesh(
        core_axis_name="c", subcore_axis_name="s",
        num_cores=NUM_CORES, num_subcores=NUM_SUBCORES),
    scratch_types=[
        pltpu.VMEM((IDX_CHUNK_ROWS * HIST,), jnp.int32),
        pltpu.VMEM((NBUF, HIST, HID), jnp.float32),
        pltpu.VMEM((ROWS_PER_W, HID), jnp.float32),
        pltpu.SemaphoreType.DMA,
    ],
    compiler_params=pltpu.CompilerParams(use_tc_tiling_on_sc=False),
)(_sc_body)


def _tail_body(s_ref, b1_ref, w2_ref, b2_ref, o_ref):
    h = jnp.maximum(s_ref[...] * (1.0 / HIST) + b1_ref[...], 0.0)
    o_ref[...] = lax.dot_general(
        h, w2_ref[...], (((1,), (1,)), ((), ())),
        precision=lax.Precision.HIGHEST) + b2_ref[...]


def _mlp_tail(s, b1, W2, b2):
    rows_blk = 2048
    return pl.pallas_call(
        _tail_body,
        grid=(BATCH // rows_blk,),
        in_specs=[
            pl.BlockSpec((rows_blk, HID), lambda i: (i, 0)),
            pl.BlockSpec((1, HID), lambda i: (0, 0)),
            pl.BlockSpec((2, HID), lambda i: (0, 0)),
            pl.BlockSpec((1, 2), lambda i: (0, 0)),
        ],
        out_specs=pl.BlockSpec((rows_blk, 2), lambda i: (i, 0)),
        out_shape=jax.ShapeDtypeStruct((BATCH, 2), jnp.float32),
    )(s, b1, W2, b2)


def kernel(x, table, W1, b1, W2, b2):
    t1 = _project_table(table, W1)
    xf = x.reshape(BATCH * HIST).astype(jnp.int32)
    s = _sc_pool(xf, t1)
    return _mlp_tail(s, b1.reshape(1, HID), W2, b2.reshape(1, 2))


# speedup vs baseline: 1.5076x; 1.5061x over previous
import functools

import numpy as np

import jax
import jax.numpy as jnp
from jax import lax
from jax.experimental import pallas as pl
from jax.experimental.pallas import tpu as pltpu
from jax.experimental.pallas import tpu_sc as plsc

VOCAB = 100000
EMBED = 128
BATCH = 16384
HIST = 200
HID = 64
LANES = 16

NUM_CORES = 2
NUM_SUBCORES = 16
NW = NUM_CORES * NUM_SUBCORES
ROWS_PER_W = BATCH // NW
IDX_CHUNK_ROWS = 128
N_CHUNKS = ROWS_PER_W // IDX_CHUNK_ROWS
NBUF = 8

_PERM = np.concatenate([
    np.arange(0, 32, 2), np.arange(1, 32, 2),
    np.arange(32, 64, 2), np.arange(33, 64, 2),
])


def _proj_body(t_ref, w_ref, o_ref):
    o_ref[...] = lax.dot_general(
        t_ref[...], w_ref[...], (((1,), (1,)), ((), ())),
        precision=lax.Precision.HIGHEST).astype(jnp.bfloat16)


def _project_table(table, W1):
    rows_blk = 1000
    return pl.pallas_call(
        _proj_body,
        grid=(VOCAB // rows_blk,),
        in_specs=[
            pl.BlockSpec((rows_blk, EMBED), lambda i: (i, 0)),
            pl.BlockSpec((HID, EMBED), lambda i: (0, 0)),
        ],
        out_specs=pl.BlockSpec((rows_blk, HID), lambda i: (i, 0)),
        out_shape=jax.ShapeDtypeStruct((VOCAB, HID), jnp.bfloat16),
    )(table, W1)


def _sc_body(xf_hbm, t1_hbm, out_hbm, idx_v, rows_v, out_v, sem):
    cid = lax.axis_index("c")
    sid = lax.axis_index("s")
    wid = sid * NUM_CORES + cid
    base = wid * ROWS_PER_W * HIST

    def load_chunk(c):
        pltpu.sync_copy(
            xf_hbm.at[pl.ds(base + c * (IDX_CHUNK_ROWS * HIST),
                            IDX_CHUNK_ROWS * HIST)],
            idx_v)

    def issue(lr, b):
        pltpu.async_copy(
            t1_hbm.at[idx_v.at[pl.ds(lr * HIST, HIST)]],
            rows_v.at[b], sem)

    def wait(b):
        pltpu.make_async_copy(
            t1_hbm.at[pl.ds(0, HIST)],
            rows_v.at[b], sem).wait()

    def accum_store(c, lr, b):
        def acc_body(i, accs):
            a0, a1, a2, a3 = accs
            r0 = 2 * i
            t0 = rows_v[b, r0, pl.ds(0, 32)] + rows_v[b, r0 + 1, pl.ds(0, 32)]
            t1 = rows_v[b, r0, pl.ds(32, 32)] + rows_v[b, r0 + 1, pl.ds(32, 32)]
            u0, u1 = plsc.unpack(t0, format=plsc.PackFormat.INTERLEAVED)
            u2, u3 = plsc.unpack(t1, format=plsc.PackFormat.INTERLEAVED)
            return (a0 + u0, a1 + u1, a2 + u2, a3 + u3)

        accs = plsc.parallel_loop(
            0, HIST // 2, 1, unroll=8,
            carry=tuple(jnp.zeros((LANES,), jnp.float32)
                        for _ in range(HID // LANES)))(acc_body)
        orow = c * IDX_CHUNK_ROWS + lr
        for k in range(HID // LANES):
            out_v[orow, pl.ds(LANES * k, LANES)] = accs[k]

    def chunk_body(c, carry):
        load_chunk(c)
        for b in range(NBUF - 1):
            issue(b, b)

        def oct_body(q, carry2):
            for b in range(NBUF):
                lr = NBUF * q + b
                wait(b)
                issue(lr + NBUF - 1, (b + NBUF - 1) % NBUF)
                accum_store(c, lr, b)
            return carry2

        lax.fori_loop(0, IDX_CHUNK_ROWS // NBUF - 1, oct_body, 0)
        tail = IDX_CHUNK_ROWS - NBUF
        wait(tail % NBUF)
        issue(IDX_CHUNK_ROWS - 1, (IDX_CHUNK_ROWS - 1) % NBUF)
        accum_store(c, tail, tail % NBUF)
        for j in range(1, NBUF):
            lr = tail + j
            wait(lr % NBUF)
            accum_store(c, lr, lr % NBUF)
        return carry

    lax.fori_loop(0, N_CHUNKS, chunk_body, 0)
    pltpu.sync_copy(out_v, out_hbm.at[pl.ds(wid * ROWS_PER_W, ROWS_PER_W)])


_sc_pool = functools.partial(
    pl.kernel,
    out_type=jax.ShapeDtypeStruct((BATCH, HID), jnp.float32),
    mesh=plsc.VectorSubcoreMesh(
        core_axis_name="c", subcore_axis_name="s",
        num_cores=NUM_CORES, num_subcores=NUM_SUBCORES),
    scratch_types=[
        pltpu.VMEM((IDX_CHUNK_ROWS * HIST,), jnp.int32),
        pltpu.VMEM((NBUF, HIST, HID), jnp.bfloat16),
        pltpu.VMEM((ROWS_PER_W, HID), jnp.float32),
        pltpu.SemaphoreType.DMA,
    ],
    compiler_params=pltpu.CompilerParams(
        use_tc_tiling_on_sc=False, needs_layout_passes=False),
)(_sc_body)


def _tail_body(s_ref, b1_ref, w2_ref, b2_ref, o_ref):
    h = jnp.maximum(s_ref[...] * (1.0 / HIST) + b1_ref[...], 0.0)
    o_ref[...] = lax.dot_general(
        h, w2_ref[...], (((1,), (1,)), ((), ())),
        precision=lax.Precision.HIGHEST) + b2_ref[...]


def _mlp_tail(s, b1, W2, b2):
    rows_blk = 2048
    return pl.pallas_call(
        _tail_body,
        grid=(BATCH // rows_blk,),
        in_specs=[
            pl.BlockSpec((rows_blk, HID), lambda i: (i, 0)),
            pl.BlockSpec((1, HID), lambda i: (0, 0)),
            pl.BlockSpec((2, HID), lambda i: (0, 0)),
            pl.BlockSpec((1, 2), lambda i: (0, 0)),
        ],
        out_specs=pl.BlockSpec((rows_blk, 2), lambda i: (i, 0)),
        out_shape=jax.ShapeDtypeStruct((BATCH, 2), jnp.float32),
    )(s, b1, W2, b2)


def kernel(x, table, W1, b1, W2, b2):
    t1 = _project_table(table, W1)
    xf = x.reshape(BATCH * HIST).astype(jnp.int32)
    s = _sc_pool(xf, t1)
    b1p = b1[_PERM].reshape(1, HID)
    w2p = W2[:, _PERM]
    return _mlp_tail(s, b1p, w2p, b2.reshape(1, 2))


# speedup vs baseline: 1.7900x; 1.1873x over previous
import functools

import numpy as np

import jax
import jax.numpy as jnp
from jax import lax
from jax.experimental import pallas as pl
from jax.experimental.pallas import tpu as pltpu
from jax.experimental.pallas import tpu_sc as plsc

VOCAB = 100000
EMBED = 128
BATCH = 16384
HIST = 200
HID = 64
LANES = 16

NUM_CORES = 2
NUM_SUBCORES = 16
NW = NUM_CORES * NUM_SUBCORES
ROWS_PER_W = BATCH // NW
IDX_CHUNK_ROWS = 128
N_CHUNKS = ROWS_PER_W // IDX_CHUNK_ROWS
NBUF = 8

_PERM = np.concatenate([
    np.arange(0, 32, 2), np.arange(1, 32, 2),
    np.arange(32, 64, 2), np.arange(33, 64, 2),
])


def _proj_body(t_ref, w_ref, o_ref):
    o_ref[...] = lax.dot_general(
        t_ref[...], w_ref[...], (((1,), (1,)), ((), ())),
        precision=lax.Precision.DEFAULT).astype(jnp.bfloat16)


def _project_table(table, W1):
    rows_blk = 2000
    return pl.pallas_call(
        _proj_body,
        grid=(VOCAB // rows_blk,),
        in_specs=[
            pl.BlockSpec((rows_blk, EMBED), lambda i: (i, 0)),
            pl.BlockSpec((HID, EMBED), lambda i: (0, 0)),
        ],
        out_specs=pl.BlockSpec((rows_blk, HID), lambda i: (i, 0)),
        out_shape=jax.ShapeDtypeStruct((VOCAB, HID), jnp.bfloat16),
    )(table, W1)


def _sc_body(xf_hbm, t1_hbm, out_hbm, idx_v, rows_v, out_v, sem):
    cid = lax.axis_index("c")
    sid = lax.axis_index("s")
    wid = sid * NUM_CORES + cid
    base = wid * ROWS_PER_W * HIST

    def load_chunk(c):
        pltpu.sync_copy(
            xf_hbm.at[pl.ds(base + c * (IDX_CHUNK_ROWS * HIST),
                            IDX_CHUNK_ROWS * HIST)],
            idx_v)

    def issue(lr, b):
        pltpu.async_copy(
            t1_hbm.at[idx_v.at[pl.ds(lr * HIST, HIST)]],
            rows_v.at[b], sem)

    def wait(b):
        pltpu.make_async_copy(
            t1_hbm.at[pl.ds(0, HIST)],
            rows_v.at[b], sem).wait()

    def accum_store(c, lr, b):
        def acc_body(i, accs):
            a0, a1, a2, a3 = accs
            r0 = 2 * i
            t0 = rows_v[b, r0, pl.ds(0, 32)] + rows_v[b, r0 + 1, pl.ds(0, 32)]
            t1 = rows_v[b, r0, pl.ds(32, 32)] + rows_v[b, r0 + 1, pl.ds(32, 32)]
            u0, u1 = plsc.unpack(t0, format=plsc.PackFormat.INTERLEAVED)
            u2, u3 = plsc.unpack(t1, format=plsc.PackFormat.INTERLEAVED)
            return (a0 + u0, a1 + u1, a2 + u2, a3 + u3)

        accs = plsc.parallel_loop(
            0, HIST // 2, 1, unroll=8,
            carry=tuple(jnp.zeros((LANES,), jnp.float32)
                        for _ in range(HID // LANES)))(acc_body)
        orow = c * IDX_CHUNK_ROWS + lr
        for k in range(HID // LANES):
            out_v[orow, pl.ds(LANES * k, LANES)] = accs[k]

    def chunk_body(c, carry):
        load_chunk(c)
        for b in range(NBUF - 1):
            issue(b, b)

        def oct_body(q, carry2):
            for b in range(NBUF):
                lr = NBUF * q + b
                wait(b)
                issue(lr + NBUF - 1, (b + NBUF - 1) % NBUF)
                accum_store(c, lr, b)
            return carry2

        lax.fori_loop(0, IDX_CHUNK_ROWS // NBUF - 1, oct_body, 0)
        tail = IDX_CHUNK_ROWS - NBUF
        wait(tail % NBUF)
        issue(IDX_CHUNK_ROWS - 1, (IDX_CHUNK_ROWS - 1) % NBUF)
        accum_store(c, tail, tail % NBUF)
        for j in range(1, NBUF):
            lr = tail + j
            wait(lr % NBUF)
            accum_store(c, lr, lr % NBUF)
        return carry

    lax.fori_loop(0, N_CHUNKS, chunk_body, 0)
    pltpu.sync_copy(out_v, out_hbm.at[pl.ds(wid * ROWS_PER_W, ROWS_PER_W)])


_sc_pool = functools.partial(
    pl.kernel,
    out_type=jax.ShapeDtypeStruct((BATCH, HID), jnp.float32),
    mesh=plsc.VectorSubcoreMesh(
        core_axis_name="c", subcore_axis_name="s",
        num_cores=NUM_CORES, num_subcores=NUM_SUBCORES),
    scratch_types=[
        pltpu.VMEM((IDX_CHUNK_ROWS * HIST,), jnp.int32),
        pltpu.VMEM((NBUF, HIST, HID), jnp.bfloat16),
        pltpu.VMEM((ROWS_PER_W, HID), jnp.float32),
        pltpu.SemaphoreType.DMA,
    ],
    compiler_params=pltpu.CompilerParams(
        use_tc_tiling_on_sc=False, needs_layout_passes=False),
)(_sc_body)


def _tail_body(s_ref, b1_ref, w2_ref, b2_ref, o_ref):
    h = jnp.maximum(s_ref[...] * (1.0 / HIST) + b1_ref[...], 0.0)
    o_ref[...] = lax.dot_general(
        h, w2_ref[...], (((1,), (1,)), ((), ())),
        precision=lax.Precision.HIGHEST) + b2_ref[...]


def _mlp_tail(s, b1, W2, b2):
    rows_blk = 2048
    return pl.pallas_call(
        _tail_body,
        grid=(BATCH // rows_blk,),
        in_specs=[
            pl.BlockSpec((rows_blk, HID), lambda i: (i, 0)),
            pl.BlockSpec((1, HID), lambda i: (0, 0)),
            pl.BlockSpec((2, HID), lambda i: (0, 0)),
            pl.BlockSpec((1, 2), lambda i: (0, 0)),
        ],
        out_specs=pl.BlockSpec((rows_blk, 2), lambda i: (i, 0)),
        out_shape=jax.ShapeDtypeStruct((BATCH, 2), jnp.float32),
    )(s, b1, W2, b2)


def kernel(x, table, W1, b1, W2, b2):
    t1 = _project_table(table, W1)
    xf = x.reshape(BATCH * HIST).astype(jnp.int32)
    s = _sc_pool(xf, t1)
    b1p = b1[_PERM].reshape(1, HID)
    w2p = W2[:, _PERM]
    return _mlp_tail(s, b1p, w2p, b2.reshape(1, 2))


# speedup vs baseline: 1.8184x; 1.0158x over previous
import functools

import numpy as np

import jax
import jax.numpy as jnp
from jax import lax
from jax.experimental import pallas as pl
from jax.experimental.pallas import tpu as pltpu
from jax.experimental.pallas import tpu_sc as plsc

VOCAB = 100000
EMBED = 128
BATCH = 16384
HIST = 200
HID = 64
LANES = 16

NUM_CORES = 2
NUM_SUBCORES = 16
NW = NUM_CORES * NUM_SUBCORES
ROWS_PER_W = BATCH // NW
IDX_CHUNK_ROWS = 128
N_CHUNKS = ROWS_PER_W // IDX_CHUNK_ROWS
NBUF = 8

_PERM = np.concatenate([
    np.arange(0, 32, 2), np.arange(1, 32, 2),
    np.arange(32, 64, 2), np.arange(33, 64, 2),
])


def _proj_body(t_ref, w_ref, o_ref):
    o_ref[...] = lax.dot_general(
        t_ref[...], w_ref[...], (((1,), (1,)), ((), ())),
        precision=lax.Precision.DEFAULT).astype(jnp.bfloat16)


def _project_table(table, W1):
    rows_blk = 2000
    return pl.pallas_call(
        _proj_body,
        grid=(VOCAB // rows_blk,),
        in_specs=[
            pl.BlockSpec((rows_blk, EMBED), lambda i: (i, 0)),
            pl.BlockSpec((HID, EMBED), lambda i: (0, 0)),
        ],
        out_specs=pl.BlockSpec((rows_blk, HID), lambda i: (i, 0)),
        out_shape=jax.ShapeDtypeStruct((VOCAB, HID), jnp.bfloat16),
    )(table, W1)


def _sc_body(xf_hbm, t1_hbm, par_hbm, out_hbm,
             idx_v, rows_v, out_v, par_v, sem):
    cid = lax.axis_index("c")
    sid = lax.axis_index("s")
    wid = sid * NUM_CORES + cid
    base = wid * ROWS_PER_W * HIST

    pltpu.sync_copy(par_hbm, par_v)
    b1v = tuple(par_v[0, pl.ds(LANES * k, LANES)] for k in range(4))
    w2v = tuple(tuple(par_v[1 + j, pl.ds(LANES * k, LANES)]
                      for k in range(4)) for j in range(2))
    b2v = tuple(par_v[3 + j, pl.ds(0, LANES)] for j in range(2))

    def load_chunk(c):
        pltpu.sync_copy(
            xf_hbm.at[pl.ds(base + c * (IDX_CHUNK_ROWS * HIST),
                            IDX_CHUNK_ROWS * HIST)],
            idx_v)

    def issue(lr, b):
        pltpu.async_copy(
            t1_hbm.at[idx_v.at[pl.ds(lr * HIST, HIST)]],
            rows_v.at[b], sem)

    def wait(b):
        pltpu.make_async_copy(
            t1_hbm.at[pl.ds(0, HIST)],
            rows_v.at[b], sem).wait()

    lane_ids = lax.iota(jnp.int32, LANES)

    def accum_store(b, ovec):
        def acc_body(i, accs):
            a0, a1, a2, a3 = accs
            r0 = 2 * i
            t0 = rows_v[b, r0, pl.ds(0, 32)] + rows_v[b, r0 + 1, pl.ds(0, 32)]
            t1 = rows_v[b, r0, pl.ds(32, 32)] + rows_v[b, r0 + 1, pl.ds(32, 32)]
            u0, u1 = plsc.unpack(t0, format=plsc.PackFormat.INTERLEAVED)
            u2, u3 = plsc.unpack(t1, format=plsc.PackFormat.INTERLEAVED)
            return (a0 + u0, a1 + u1, a2 + u2, a3 + u3)

        accs = plsc.parallel_loop(
            0, HIST // 2, 1, unroll=8,
            carry=tuple(jnp.zeros((LANES,), jnp.float32)
                        for _ in range(HID // LANES)))(acc_body)
        h = tuple(jnp.maximum(accs[k] * (1.0 / HIST) + b1v[k], 0.0)
                  for k in range(4))
        for j in range(2):
            t = (h[0] * w2v[j][0] + h[1] * w2v[j][1]
                 + h[2] * w2v[j][2] + h[3] * w2v[j][3] + b2v[j])
            o = jnp.sum(t)
            ovec = jnp.where(lane_ids == 2 * b + j,
                             jnp.full((LANES,), o), ovec)
        return ovec

    def chunk_body(c, carry):
        load_chunk(c)
        for b in range(NBUF - 1):
            issue(b, b)

        def oct_body(q, carry2):
            ovec = jnp.zeros((LANES,), jnp.float32)
            for b in range(NBUF):
                lr = NBUF * q + b
                wait(b)
                issue(lr + NBUF - 1, (b + NBUF - 1) % NBUF)
                ovec = accum_store(b, ovec)
            g = c * (IDX_CHUNK_ROWS // NBUF) + q
            out_v[pl.ds(LANES * g, LANES)] = ovec
            return carry2

        lax.fori_loop(0, IDX_CHUNK_ROWS // NBUF - 1, oct_body, 0)
        tail = IDX_CHUNK_ROWS - NBUF
        ovec = jnp.zeros((LANES,), jnp.float32)
        wait(0)
        issue(IDX_CHUNK_ROWS - 1, NBUF - 1)
        ovec = accum_store(0, ovec)
        for b in range(1, NBUF):
            wait(b)
            ovec = accum_store(b, ovec)
        g = c * (IDX_CHUNK_ROWS // NBUF) + (IDX_CHUNK_ROWS // NBUF - 1)
        out_v[pl.ds(LANES * g, LANES)] = ovec
        return carry

    lax.fori_loop(0, N_CHUNKS, chunk_body, 0)
    pltpu.sync_copy(out_v, out_hbm.at[pl.ds(wid * ROWS_PER_W * 2,
                                            ROWS_PER_W * 2)])


_sc_pool = functools.partial(
    pl.kernel,
    out_type=jax.ShapeDtypeStruct((BATCH * 2,), jnp.float32),
    mesh=plsc.VectorSubcoreMesh(
        core_axis_name="c", subcore_axis_name="s",
        num_cores=NUM_CORES, num_subcores=NUM_SUBCORES),
    scratch_types=[
        pltpu.VMEM((IDX_CHUNK_ROWS * HIST,), jnp.int32),
        pltpu.VMEM((NBUF, HIST, HID), jnp.bfloat16),
        pltpu.VMEM((ROWS_PER_W * 2,), jnp.float32),
        pltpu.VMEM((5, HID), jnp.float32),
        pltpu.SemaphoreType.DMA,
    ],
    compiler_params=pltpu.CompilerParams(
        use_tc_tiling_on_sc=False, needs_layout_passes=False),
)(_sc_body)


def kernel(x, table, W1, b1, W2, b2):
    t1 = _project_table(table, W1)
    xf = x.reshape(BATCH * HIST).astype(jnp.int32)
    par = jnp.zeros((5, HID), jnp.float32)
    par = par.at[0].set(b1[_PERM])
    par = par.at[1:3].set(W2[:, _PERM])
    par = par.at[3, 0].set(b2[0])
    par = par.at[4, 0].set(b2[1])
    return _sc_pool(xf, t1, par).reshape(BATCH, 2)
